# baseline (speedup 1.0000x reference)
import functools

import jax
import jax.numpy as jnp
from jax import lax
from jax.experimental import pallas as pl
from jax.experimental.pallas import tpu as pltpu
from jax.experimental.pallas import tpu_sc as plsc

_DT = 0.001
_TAU_SYN_INV = 200.0
_TAU_MEM_INV = 100.0
_V_TH = 0.5

_B, _S = 16, 2048
_IN_F, _HID_F, _OUT_F = 1024, 4096, 256
_T = 31
_ZWORDS = _T * _IN_F


def _sc_scatter(x):
    mesh = plsc.VectorSubcoreMesh(core_axis_name="c", subcore_axis_name="s")

    @functools.partial(
        pl.kernel,
        mesh=mesh,
        compiler_params=pltpu.CompilerParams(needs_layout_passes=False),
        out_type=jax.ShapeDtypeStruct((_T, _B, _IN_F), jnp.float32),
        scratch_types=[
            pltpu.VMEM((_S * 4,), jnp.float32),
            pltpu.VMEM((_T, _IN_F), jnp.float32),
        ],
    )
    def k(x_hbm, out_hbm, xloc, zloc):
        cid = lax.axis_index("c")
        sid = lax.axis_index("s")
        wid = sid * 2 + cid

        @pl.when(wid < _B)
        def _():
            b = wid
            pltpu.sync_copy(x_hbm.at[b], xloc)

            zero16 = jnp.zeros((16,), jnp.float32)

            def zero_row(r, carry):
                for c in range(_IN_F // 16):
                    zloc[r, pl.ds(c * 16, 16)] = zero16
                return carry

            lax.fori_loop(0, _T, zero_row, 0)

            lanes = lax.iota(jnp.int32, 16)
            nxt_idx = jnp.minimum(lanes + 1, 15)
            gdn = lax.GatherDimensionNumbers(
                offset_dims=(), collapsed_slice_dims=(0,), start_index_map=(0,)
            )

            def chunk(ci, carry):
                base = (ci * 16 + lanes) * 4
                e0 = plsc.load_gather(xloc, [base])
                e1 = plsc.load_gather(xloc, [base + 1])
                ev = plsc.load_gather(xloc, [base + 2])
                et = plsc.load_gather(xloc, [base + 3])
                feat = (e0 * e1).astype(jnp.int32)
                ts = et.astype(jnp.int32)
                key = ts * _IN_F + feat
                sortk = key * 16 + lanes
                sk, sv = plsc.sort_key_val(sortk, ev)
                key_s = lax.shift_right_logical(sk, 4)
                nxt = lax.gather(
                    key_s,
                    nxt_idx[:, None],
                    gdn,
                    (1,),
                    mode=lax.GatherScatterMode.PROMISE_IN_BOUNDS,
                )
                keep = ((key_s != nxt) | (lanes == 15)) & (key_s < _ZWORDS)
                ts_s = lax.shift_right_logical(key_s, 10)
                ft_s = lax.bitwise_and(key_s, _IN_F - 1)
                plsc.store_scatter(zloc, [ts_s, ft_s], sv, mask=keep)
                return carry

            lax.fori_loop(0, _S // 16, chunk, 0)
            pltpu.sync_copy(zloc, out_hbm.at[:, b])

    return k(x.reshape(_B, _S * 4))


_NCHUNK = 32
_CROWS = _HID_F // _NCHUNK
_WICHUNK = 16
_WIROWS = _HID_F // _WICHUNK


def _rnn_kernel(
    zt_ref,
    w_in_hbm,
    w_rec_hbm,
    w_out_ref,
    out_ref,
    i_in_s,
    wrec16,
    zh_s,
    z_s,
    v_s,
    i_s,
    y_s,
    vo_s,
    io_s,
    wtmp0,
    wtmp1,
    witmp0,
    witmp1,
    wch16,
    sem0,
    sem1,
    wisem0,
    wisem1,
):
    bufs = (wtmp0, wtmp1)
    sems = (sem0, sem1)
    wibufs = (witmp0, witmp1)
    wisems = (wisem0, wisem1)
    pltpu.make_async_copy(w_in_hbm.at[pl.ds(0, _WIROWS), :], wibufs[0], wisems[0]).start()
    pltpu.make_async_copy(w_in_hbm.at[pl.ds(_WIROWS, _WIROWS), :], wibufs[1], wisems[1]).start()
    pltpu.make_async_copy(w_rec_hbm.at[pl.ds(0, _CROWS), :], bufs[0], sems[0]).start()
    pltpu.make_async_copy(w_rec_hbm.at[pl.ds(_CROWS, _CROWS), :], bufs[1], sems[1]).start()

    zt16 = zt_ref[...].astype(jnp.bfloat16)
    for k in range(_WICHUNK):
        pltpu.make_async_copy(
            w_in_hbm.at[pl.ds(k * _WIROWS, _WIROWS), :], wibufs[k % 2], wisems[k % 2]
        ).wait()
        wch16[...] = wibufs[k % 2][...].astype(jnp.bfloat16)
        if k + 2 < _WICHUNK:
            pltpu.make_async_copy(
                w_in_hbm.at[pl.ds((k + 2) * _WIROWS, _WIROWS), :],
                wibufs[k % 2],
                wisems[k % 2],
            ).start()
        i_in_s[:, pl.ds(k * _WIROWS, _WIROWS)] = lax.dot_general(
            zt16,
            wch16[...],
            (((1,), (1,)), ((), ())),
            preferred_element_type=jnp.float32,
        )

    z_s[...] = jnp.zeros_like(z_s)
    v_s[...] = jnp.zeros_like(v_s)
    i_s[...] = jnp.zeros_like(i_s)

    def warm_step(t):
        v = v_s[...]
        i = i_s[...]
        v_dec = v + _DT * _TAU_MEM_INV * ((0.0 - v) + i)
        i_dec = i - _DT * _TAU_SYN_INV * i
        z_new = ((v_dec - _V_TH) > 0.0).astype(jnp.float32)
        v_s[...] = (1.0 - z_new) * v_dec
        i_s[...] = i_dec + i_in_s[pl.ds(t * _B, _B), :]
        z16 = z_new.astype(jnp.bfloat16)
        z_s[...] = z16
        zh_s[pl.ds(t * _B, _B), :] = z16

    warm_step(0)
    warm_step(1)

    for k in range(_NCHUNK):
        pltpu.make_async_copy(
            w_rec_hbm.at[pl.ds(k * _CROWS, _CROWS), :], bufs[k % 2], sems[k % 2]
        ).wait()
        wrec16[pl.ds(k * _CROWS, _CROWS), :] = bufs[k % 2][...].astype(jnp.bfloat16)
        if k + 2 < _NCHUNK:
            pltpu.make_async_copy(
                w_rec_hbm.at[pl.ds((k + 2) * _CROWS, _CROWS), :],
                bufs[k % 2],
                sems[k % 2],
            ).start()

    def step(t, carry):
        v = v_s[...]
        i = i_s[...]
        v_dec = v + _DT * _TAU_MEM_INV * ((0.0 - v) + i)
        i_dec = i - _DT * _TAU_SYN_INV * i
        z_new = ((v_dec - _V_TH) > 0.0).astype(jnp.float32)
        v_s[...] = (1.0 - z_new) * v_dec
        rec = lax.dot_general(
            z_s[...],
            wrec16[...],
            (((1,), (1,)), ((), ())),
            preferred_element_type=jnp.float32,
        )
        i_s[...] = (i_dec + i_in_s[pl.ds(t * _B, _B), :]) + rec
        z16 = z_new.astype(jnp.bfloat16)
        z_s[...] = z16
        zh_s[pl.ds(t * _B, _B), :] = z16
        return carry

    lax.fori_loop(2, _T, step, 0)

    y_s[...] = lax.dot_general(
        zh_s[...],
        w_out_ref[...],
        (((1,), (1,)), ((), ())),
        preferred_element_type=jnp.float32,
    )

    vo_s[...] = jnp.zeros_like(vo_s)
    io_s[...] = jnp.zeros_like(io_s)

    def li_step(t, carry):
        vo = vo_s[...]
        io = io_s[...]
        y = y_s[pl.ds(t * _B, _B), :]
        vo_new = vo + _DT * _TAU_MEM_INV * ((0.0 - vo) + io)
        io_s[...] = io - _DT * _TAU_SYN_INV * io + y
        vo_s[...] = vo_new
        out_ref[pl.ds(t * _B, _B), :] = vo_new
        return carry

    lax.fori_loop(0, _T, li_step, 0)


def kernel(x, W_in, W_rec, W_out):
    z = _sc_scatter(x)
    zt = z.reshape(_T * _B, _IN_F)
    v_out = pl.pallas_call(
        _rnn_kernel,
        out_shape=jax.ShapeDtypeStruct((_T * _B, _OUT_F), jnp.float32),
        in_specs=[
            pl.BlockSpec(memory_space=pltpu.MemorySpace.VMEM),
            pl.BlockSpec(memory_space=pltpu.MemorySpace.HBM),
            pl.BlockSpec(memory_space=pltpu.MemorySpace.HBM),
            pl.BlockSpec(memory_space=pltpu.MemorySpace.VMEM),
        ],
        scratch_shapes=[
            pltpu.VMEM((_T * _B, _HID_F), jnp.float32),
            pltpu.VMEM((_HID_F, _HID_F), jnp.bfloat16),
            pltpu.VMEM((_T * _B, _HID_F), jnp.bfloat16),
            pltpu.VMEM((_B, _HID_F), jnp.bfloat16),
            pltpu.VMEM((_B, _HID_F), jnp.float32),
            pltpu.VMEM((_B, _HID_F), jnp.float32),
            pltpu.VMEM((_T * _B, _OUT_F), jnp.float32),
            pltpu.VMEM((_B, _OUT_F), jnp.float32),
            pltpu.VMEM((_B, _OUT_F), jnp.float32),
            pltpu.VMEM((_CROWS, _HID_F), jnp.float32),
            pltpu.VMEM((_CROWS, _HID_F), jnp.float32),
            pltpu.VMEM((_WIROWS, _IN_F), jnp.float32),
            pltpu.VMEM((_WIROWS, _IN_F), jnp.float32),
            pltpu.VMEM((_WIROWS, _IN_F), jnp.bfloat16),
            pltpu.SemaphoreType.DMA,
            pltpu.SemaphoreType.DMA,
            pltpu.SemaphoreType.DMA,
            pltpu.SemaphoreType.DMA,
        ],
        compiler_params=pltpu.CompilerParams(vmem_limit_bytes=58 * 1024 * 1024),
    )(zt, W_in, W_rec, W_out.astype(jnp.bfloat16))
    return v_out.reshape(_T, _B, _OUT_F)


# speedup vs baseline: 1.0350x; 1.0350x over previous
import functools

import jax
import jax.numpy as jnp
from jax import lax
from jax.experimental import pallas as pl
from jax.experimental.pallas import tpu as pltpu
from jax.experimental.pallas import tpu_sc as plsc

_DT = 0.001
_TAU_SYN_INV = 200.0
_TAU_MEM_INV = 100.0
_V_TH = 0.5

_B, _S = 16, 2048
_IN_F, _HID_F, _OUT_F = 1024, 4096, 256
_T = 31
_ZWORDS = _T * _IN_F


def _sc_scatter(x):
    mesh = plsc.VectorSubcoreMesh(core_axis_name="c", subcore_axis_name="s")

    @functools.partial(
        pl.kernel,
        mesh=mesh,
        compiler_params=pltpu.CompilerParams(needs_layout_passes=False),
        out_type=jax.ShapeDtypeStruct((_T, _B, _IN_F), jnp.float32),
        scratch_types=[
            pltpu.VMEM((_S * 4,), jnp.float32),
            pltpu.VMEM((_T, _IN_F), jnp.float32),
        ],
    )
    def k(x_hbm, out_hbm, xloc, zloc):
        cid = lax.axis_index("c")
        sid = lax.axis_index("s")
        wid = sid * 2 + cid

        @pl.when(wid < _B)
        def _():
            b = wid
            pltpu.sync_copy(x_hbm.at[b], xloc)

            zero16 = jnp.zeros((16,), jnp.float32)

            def zero_row(r, carry):
                for c in range(_IN_F // 16):
                    zloc[r, pl.ds(c * 16, 16)] = zero16
                return carry

            lax.fori_loop(0, _T, zero_row, 0)

            lanes = lax.iota(jnp.int32, 16)
            nxt_idx = jnp.minimum(lanes + 1, 15)
            gdn = lax.GatherDimensionNumbers(
                offset_dims=(), collapsed_slice_dims=(0,), start_index_map=(0,)
            )

            def chunk(ci, carry):
                base = (ci * 16 + lanes) * 4
                e0 = plsc.load_gather(xloc, [base])
                e1 = plsc.load_gather(xloc, [base + 1])
                ev = plsc.load_gather(xloc, [base + 2])
                et = plsc.load_gather(xloc, [base + 3])
                feat = (e0 * e1).astype(jnp.int32)
                ts = et.astype(jnp.int32)
                key = ts * _IN_F + feat
                sortk = key * 16 + lanes
                sk, sv = plsc.sort_key_val(sortk, ev)
                key_s = lax.shift_right_logical(sk, 4)
                nxt = lax.gather(
                    key_s,
                    nxt_idx[:, None],
                    gdn,
                    (1,),
                    mode=lax.GatherScatterMode.PROMISE_IN_BOUNDS,
                )
                keep = ((key_s != nxt) | (lanes == 15)) & (key_s < _ZWORDS)
                ts_s = lax.shift_right_logical(key_s, 10)
                ft_s = lax.bitwise_and(key_s, _IN_F - 1)
                plsc.store_scatter(zloc, [ts_s, ft_s], sv, mask=keep)
                return carry

            lax.fori_loop(0, _S // 16, chunk, 0)
            pltpu.sync_copy(zloc, out_hbm.at[:, b])

    return k(x.reshape(_B, _S * 4))


_NCHUNK = 32
_CROWS = _HID_F // _NCHUNK
_WICHUNK = 16
_WIROWS = _HID_F // _WICHUNK


def _rnn_kernel(
    zt_ref,
    w_in_hbm,
    w_rec_hbm,
    w_out_ref,
    out_ref,
    i_in_s,
    wrec16,
    zh_s,
    z_s,
    v_s,
    i_s,
    y_s,
    vo_s,
    io_s,
    wtmp0,
    wtmp1,
    witmp0,
    witmp1,
    wch16,
    sem0,
    sem1,
    wisem0,
    wisem1,
):
    bufs = (wtmp0, wtmp1)
    sems = (sem0, sem1)
    wibufs = (witmp0, witmp1)
    wisems = (wisem0, wisem1)
    pltpu.make_async_copy(w_in_hbm.at[pl.ds(0, _WIROWS), :], wibufs[0], wisems[0]).start()
    pltpu.make_async_copy(w_in_hbm.at[pl.ds(_WIROWS, _WIROWS), :], wibufs[1], wisems[1]).start()
    pltpu.make_async_copy(w_rec_hbm.at[pl.ds(0, _CROWS), :], bufs[0], sems[0]).start()
    pltpu.make_async_copy(w_rec_hbm.at[pl.ds(_CROWS, _CROWS), :], bufs[1], sems[1]).start()

    def wrec_chunk(k):
        pltpu.make_async_copy(
            w_rec_hbm.at[pl.ds(k * _CROWS, _CROWS), :], bufs[k % 2], sems[k % 2]
        ).wait()
        wrec16[pl.ds(k * _CROWS, _CROWS), :] = bufs[k % 2][...].astype(jnp.bfloat16)
        if k + 2 < _NCHUNK:
            pltpu.make_async_copy(
                w_rec_hbm.at[pl.ds((k + 2) * _CROWS, _CROWS), :],
                bufs[k % 2],
                sems[k % 2],
            ).start()

    zt16 = zt_ref[...].astype(jnp.bfloat16)
    for k in range(_WICHUNK):
        pltpu.make_async_copy(
            w_in_hbm.at[pl.ds(k * _WIROWS, _WIROWS), :], wibufs[k % 2], wisems[k % 2]
        ).wait()
        wch16[...] = wibufs[k % 2][...].astype(jnp.bfloat16)
        if k + 2 < _WICHUNK:
            pltpu.make_async_copy(
                w_in_hbm.at[pl.ds((k + 2) * _WIROWS, _WIROWS), :],
                wibufs[k % 2],
                wisems[k % 2],
            ).start()
        i_in_s[:, pl.ds(k * _WIROWS, _WIROWS)] = lax.dot_general(
            zt16,
            wch16[...],
            (((1,), (1,)), ((), ())),
            preferred_element_type=jnp.float32,
        )
        wrec_chunk(k)

    z_s[...] = jnp.zeros_like(z_s)
    v_s[...] = jnp.zeros_like(v_s)
    i_s[...] = jnp.zeros_like(i_s)

    def warm_step(t):
        v = v_s[...]
        i = i_s[...]
        v_dec = v + _DT * _TAU_MEM_INV * ((0.0 - v) + i)
        i_dec = i - _DT * _TAU_SYN_INV * i
        z_new = ((v_dec - _V_TH) > 0.0).astype(jnp.float32)
        v_s[...] = (1.0 - z_new) * v_dec
        i_s[...] = i_dec + i_in_s[pl.ds(t * _B, _B), :]
        z16 = z_new.astype(jnp.bfloat16)
        z_s[...] = z16
        zh_s[pl.ds(t * _B, _B), :] = z16

    warm_step(0)
    warm_step(1)

    for k in range(_WICHUNK, _NCHUNK):
        wrec_chunk(k)

    def step(t, carry):
        v = v_s[...]
        i = i_s[...]
        v_dec = v + _DT * _TAU_MEM_INV * ((0.0 - v) + i)
        i_dec = i - _DT * _TAU_SYN_INV * i
        z_new = ((v_dec - _V_TH) > 0.0).astype(jnp.float32)
        v_s[...] = (1.0 - z_new) * v_dec
        rec = lax.dot_general(
            z_s[...],
            wrec16[...],
            (((1,), (1,)), ((), ())),
            preferred_element_type=jnp.float32,
        )
        i_s[...] = (i_dec + i_in_s[pl.ds(t * _B, _B), :]) + rec
        z16 = z_new.astype(jnp.bfloat16)
        z_s[...] = z16
        zh_s[pl.ds(t * _B, _B), :] = z16
        return carry

    lax.fori_loop(2, _T, step, 0)

    y_s[...] = lax.dot_general(
        zh_s[...],
        w_out_ref[...],
        (((1,), (1,)), ((), ())),
        preferred_element_type=jnp.float32,
    )

    vo_s[...] = jnp.zeros_like(vo_s)
    io_s[...] = jnp.zeros_like(io_s)

    def li_step(t, carry):
        vo = vo_s[...]
        io = io_s[...]
        y = y_s[pl.ds(t * _B, _B), :]
        vo_new = vo + _DT * _TAU_MEM_INV * ((0.0 - vo) + io)
        io_s[...] = io - _DT * _TAU_SYN_INV * io + y
        vo_s[...] = vo_new
        out_ref[pl.ds(t * _B, _B), :] = vo_new
        return carry

    lax.fori_loop(0, _T, li_step, 0)


def kernel(x, W_in, W_rec, W_out):
    z = _sc_scatter(x)
    zt = z.reshape(_T * _B, _IN_F)
    v_out = pl.pallas_call(
        _rnn_kernel,
        out_shape=jax.ShapeDtypeStruct((_T * _B, _OUT_F), jnp.float32),
        in_specs=[
            pl.BlockSpec(memory_space=pltpu.MemorySpace.VMEM),
            pl.BlockSpec(memory_space=pltpu.MemorySpace.HBM),
            pl.BlockSpec(memory_space=pltpu.MemorySpace.HBM),
            pl.BlockSpec(memory_space=pltpu.MemorySpace.VMEM),
        ],
        scratch_shapes=[
            pltpu.VMEM((_T * _B, _HID_F), jnp.float32),
            pltpu.VMEM((_HID_F, _HID_F), jnp.bfloat16),
            pltpu.VMEM((_T * _B, _HID_F), jnp.bfloat16),
            pltpu.VMEM((_B, _HID_F), jnp.bfloat16),
            pltpu.VMEM((_B, _HID_F), jnp.float32),
            pltpu.VMEM((_B, _HID_F), jnp.float32),
            pltpu.VMEM((_T * _B, _OUT_F), jnp.float32),
            pltpu.VMEM((_B, _OUT_F), jnp.float32),
            pltpu.VMEM((_B, _OUT_F), jnp.float32),
            pltpu.VMEM((_CROWS, _HID_F), jnp.float32),
            pltpu.VMEM((_CROWS, _HID_F), jnp.float32),
            pltpu.VMEM((_WIROWS, _IN_F), jnp.float32),
            pltpu.VMEM((_WIROWS, _IN_F), jnp.float32),
            pltpu.VMEM((_WIROWS, _IN_F), jnp.bfloat16),
            pltpu.SemaphoreType.DMA,
            pltpu.SemaphoreType.DMA,
            pltpu.SemaphoreType.DMA,
            pltpu.SemaphoreType.DMA,
        ],
        compiler_params=pltpu.CompilerParams(vmem_limit_bytes=58 * 1024 * 1024),
    )(zt, W_in, W_rec, W_out.astype(jnp.bfloat16))
    return v_out.reshape(_T, _B, _OUT_F)


# speedup vs baseline: 1.0350x; 1.0000x over previous
import functools

import jax
import jax.numpy as jnp
from jax import lax
from jax.experimental import pallas as pl
from jax.experimental.pallas import tpu as pltpu
from jax.experimental.pallas import tpu_sc as plsc

_DT = 0.001
_TAU_SYN_INV = 200.0
_TAU_MEM_INV = 100.0
_V_TH = 0.5

_B, _S = 16, 2048
_IN_F, _HID_F, _OUT_F = 1024, 4096, 256
_T = 31
_ZWORDS = _T * _IN_F


def _sc_scatter(x):
    mesh = plsc.VectorSubcoreMesh(core_axis_name="c", subcore_axis_name="s")

    @functools.partial(
        pl.kernel,
        mesh=mesh,
        compiler_params=pltpu.CompilerParams(needs_layout_passes=False),
        out_type=jax.ShapeDtypeStruct((_T, _B, _IN_F), jnp.float32),
        scratch_types=[
            pltpu.VMEM((_S * 4,), jnp.float32),
            pltpu.VMEM((_T, _IN_F), jnp.float32),
        ],
    )
    def k(x_hbm, out_hbm, xloc, zloc):
        cid = lax.axis_index("c")
        sid = lax.axis_index("s")
        wid = sid * 2 + cid

        @pl.when(wid < _B)
        def _():
            b = wid
            pltpu.sync_copy(x_hbm.at[b], xloc)

            zero16 = jnp.zeros((16,), jnp.float32)

            def zero_row(r, carry):
                for c in range(_IN_F // 16):
                    zloc[r, pl.ds(c * 16, 16)] = zero16
                return carry

            lax.fori_loop(0, _T, zero_row, 0)

            lanes = lax.iota(jnp.int32, 16)
            nxt_idx = jnp.minimum(lanes + 1, 15)
            gdn = lax.GatherDimensionNumbers(
                offset_dims=(), collapsed_slice_dims=(0,), start_index_map=(0,)
            )

            def chunk(ci, carry):
                base = (ci * 16 + lanes) * 4
                e0 = plsc.load_gather(xloc, [base])
                e1 = plsc.load_gather(xloc, [base + 1])
                ev = plsc.load_gather(xloc, [base + 2])
                et = plsc.load_gather(xloc, [base + 3])
                feat = (e0 * e1).astype(jnp.int32)
                ts = et.astype(jnp.int32)
                key = ts * _IN_F + feat
                sortk = key * 16 + lanes
                sk, sv = plsc.sort_key_val(sortk, ev)
                key_s = lax.shift_right_logical(sk, 4)
                nxt = lax.gather(
                    key_s,
                    nxt_idx[:, None],
                    gdn,
                    (1,),
                    mode=lax.GatherScatterMode.PROMISE_IN_BOUNDS,
                )
                keep = ((key_s != nxt) | (lanes == 15)) & (key_s < _ZWORDS)
                ts_s = lax.shift_right_logical(key_s, 10)
                ft_s = lax.bitwise_and(key_s, _IN_F - 1)
                plsc.store_scatter(zloc, [ts_s, ft_s], sv, mask=keep)
                return carry

            lax.fori_loop(0, _S // 16, chunk, 0)
            pltpu.sync_copy(zloc, out_hbm.at[:, b])

    return k(x.reshape(_B, _S * 4))


_NCHUNK = 32
_CROWS = _HID_F // _NCHUNK
_WICHUNK = 16
_WIROWS = _HID_F // _WICHUNK


def _rnn_kernel(
    zt_ref,
    w_in_hbm,
    w_rec_hbm,
    w_out_ref,
    out_ref,
    i_in_s,
    wrec16,
    zh_s,
    z_s,
    v_s,
    i_s,
    y_s,
    vo_s,
    io_s,
    wtmp0,
    wtmp1,
    witmp0,
    witmp1,
    wch16,
    zt16_s,
    sem0,
    sem1,
    wisem0,
    wisem1,
):
    bufs = (wtmp0, wtmp1)
    sems = (sem0, sem1)
    wibufs = (witmp0, witmp1)
    wisems = (wisem0, wisem1)
    pltpu.make_async_copy(w_in_hbm.at[pl.ds(0, _WIROWS), :], wibufs[0], wisems[0]).start()
    pltpu.make_async_copy(w_in_hbm.at[pl.ds(_WIROWS, _WIROWS), :], wibufs[1], wisems[1]).start()
    pltpu.make_async_copy(w_rec_hbm.at[pl.ds(0, _CROWS), :], bufs[0], sems[0]).start()
    pltpu.make_async_copy(w_rec_hbm.at[pl.ds(_CROWS, _CROWS), :], bufs[1], sems[1]).start()

    def wrec_chunk(k):
        pltpu.make_async_copy(
            w_rec_hbm.at[pl.ds(k * _CROWS, _CROWS), :], bufs[k % 2], sems[k % 2]
        ).wait()
        wrec16[pl.ds(k * _CROWS, _CROWS), :] = bufs[k % 2][...].astype(jnp.bfloat16)
        if k + 2 < _NCHUNK:
            pltpu.make_async_copy(
                w_rec_hbm.at[pl.ds((k + 2) * _CROWS, _CROWS), :],
                bufs[k % 2],
                sems[k % 2],
            ).start()

    zt16_s[...] = zt_ref[...].astype(jnp.bfloat16)
    for k in range(_WICHUNK):
        pltpu.make_async_copy(
            w_in_hbm.at[pl.ds(k * _WIROWS, _WIROWS), :], wibufs[k % 2], wisems[k % 2]
        ).wait()
        wch16[...] = wibufs[k % 2][...].astype(jnp.bfloat16)
        if k + 2 < _WICHUNK:
            pltpu.make_async_copy(
                w_in_hbm.at[pl.ds((k + 2) * _WIROWS, _WIROWS), :],
                wibufs[k % 2],
                wisems[k % 2],
            ).start()
        i_in_s[:, pl.ds(k * _WIROWS, _WIROWS)] = lax.dot_general(
            zt16_s[...],
            wch16[...],
            (((1,), (1,)), ((), ())),
            preferred_element_type=jnp.float32,
        )
        wrec_chunk(k)

    z_s[...] = jnp.zeros_like(z_s)
    v_s[...] = jnp.zeros_like(v_s)
    i_s[...] = jnp.zeros_like(i_s)

    def warm_step(t):
        v = v_s[...]
        i = i_s[...]
        v_dec = v + _DT * _TAU_MEM_INV * ((0.0 - v) + i)
        i_dec = i - _DT * _TAU_SYN_INV * i
        z_new = ((v_dec - _V_TH) > 0.0).astype(jnp.float32)
        v_s[...] = (1.0 - z_new) * v_dec
        i_s[...] = i_dec + i_in_s[pl.ds(t * _B, _B), :]
        z16 = z_new.astype(jnp.bfloat16)
        z_s[...] = z16
        zh_s[pl.ds(t * _B, _B), :] = z16

    warm_step(0)
    warm_step(1)

    for k in range(_WICHUNK, _NCHUNK):
        wrec_chunk(k)

    def step(t, carry):
        v = v_s[...]
        i = i_s[...]
        v_dec = v + _DT * _TAU_MEM_INV * ((0.0 - v) + i)
        i_dec = i - _DT * _TAU_SYN_INV * i
        z_new = ((v_dec - _V_TH) > 0.0).astype(jnp.float32)
        v_s[...] = (1.0 - z_new) * v_dec
        rec = lax.dot_general(
            z_s[...],
            wrec16[...],
            (((1,), (1,)), ((), ())),
            preferred_element_type=jnp.float32,
        )
        i_s[...] = (i_dec + i_in_s[pl.ds(t * _B, _B), :]) + rec
        z16 = z_new.astype(jnp.bfloat16)
        z_s[...] = z16
        zh_s[pl.ds(t * _B, _B), :] = z16
        return carry

    lax.fori_loop(2, _T, step, 0)

    y_s[...] = lax.dot_general(
        zh_s[...],
        w_out_ref[...],
        (((1,), (1,)), ((), ())),
        preferred_element_type=jnp.float32,
    )

    vo_s[...] = jnp.zeros_like(vo_s)
    io_s[...] = jnp.zeros_like(io_s)

    def li_step(t, carry):
        vo = vo_s[...]
        io = io_s[...]
        y = y_s[pl.ds(t * _B, _B), :]
        vo_new = vo + _DT * _TAU_MEM_INV * ((0.0 - vo) + io)
        io_s[...] = io - _DT * _TAU_SYN_INV * io + y
        vo_s[...] = vo_new
        out_ref[pl.ds(t * _B, _B), :] = vo_new
        return carry

    lax.fori_loop(0, _T, li_step, 0)


def kernel(x, W_in, W_rec, W_out):
    z = _sc_scatter(x)
    zt = z.reshape(_T * _B, _IN_F)
    v_out = pl.pallas_call(
        _rnn_kernel,
        out_shape=jax.ShapeDtypeStruct((_T * _B, _OUT_F), jnp.float32),
        in_specs=[
            pl.BlockSpec(memory_space=pltpu.MemorySpace.VMEM),
            pl.BlockSpec(memory_space=pltpu.MemorySpace.HBM),
            pl.BlockSpec(memory_space=pltpu.MemorySpace.HBM),
            pl.BlockSpec(memory_space=pltpu.MemorySpace.VMEM),
        ],
        scratch_shapes=[
            pltpu.VMEM((_T * _B, _HID_F), jnp.float32),
            pltpu.VMEM((_HID_F, _HID_F), jnp.bfloat16),
            pltpu.VMEM((_T * _B, _HID_F), jnp.bfloat16),
            pltpu.VMEM((_B, _HID_F), jnp.bfloat16),
            pltpu.VMEM((_B, _HID_F), jnp.float32),
            pltpu.VMEM((_B, _HID_F), jnp.float32),
            pltpu.VMEM((_T * _B, _OUT_F), jnp.float32),
            pltpu.VMEM((_B, _OUT_F), jnp.float32),
            pltpu.VMEM((_B, _OUT_F), jnp.float32),
            pltpu.VMEM((_CROWS, _HID_F), jnp.float32),
            pltpu.VMEM((_CROWS, _HID_F), jnp.float32),
            pltpu.VMEM((_WIROWS, _IN_F), jnp.float32),
            pltpu.VMEM((_WIROWS, _IN_F), jnp.float32),
            pltpu.VMEM((_WIROWS, _IN_F), jnp.bfloat16),
            pltpu.VMEM((_T * _B, _IN_F), jnp.bfloat16),
            pltpu.SemaphoreType.DMA,
            pltpu.SemaphoreType.DMA,
            pltpu.SemaphoreType.DMA,
            pltpu.SemaphoreType.DMA,
        ],
        compiler_params=pltpu.CompilerParams(vmem_limit_bytes=58 * 1024 * 1024),
    )(zt, W_in, W_rec, W_out.astype(jnp.bfloat16))
    return v_out.reshape(_T, _B, _OUT_F)


# speedup vs baseline: 1.3561x; 1.3102x over previous
import functools

import jax
import jax.numpy as jnp
from jax import lax
from jax.experimental import pallas as pl
from jax.experimental.pallas import tpu as pltpu
from jax.experimental.pallas import tpu_sc as plsc

_DT = 0.001
_TAU_SYN_INV = 200.0
_TAU_MEM_INV = 100.0
_V_TH = 0.5

_B, _S = 16, 2048
_IN_F, _HID_F, _OUT_F = 1024, 4096, 256
_T = 31
_ZWORDS = _T * _IN_F


def _sc_scatter(x):
    mesh = plsc.VectorSubcoreMesh(core_axis_name="c", subcore_axis_name="s")

    @functools.partial(
        pl.kernel,
        mesh=mesh,
        compiler_params=pltpu.CompilerParams(needs_layout_passes=False),
        out_type=jax.ShapeDtypeStruct((_T, _B, _IN_F), jnp.float32),
        scratch_types=[
            pltpu.VMEM((_S * 4,), jnp.float32),
            pltpu.VMEM((_T, _IN_F), jnp.float32),
        ],
    )
    def k(x_hbm, out_hbm, xloc, zloc):
        cid = lax.axis_index("c")
        sid = lax.axis_index("s")
        wid = sid * 2 + cid

        @pl.when(wid < _B)
        def _():
            b = wid
            pltpu.sync_copy(x_hbm.at[b], xloc)

            zero16 = jnp.zeros((16,), jnp.float32)

            def zero_row(r, carry):
                for c in range(_IN_F // 16):
                    zloc[r, pl.ds(c * 16, 16)] = zero16
                return carry

            lax.fori_loop(0, _T, zero_row, 0)

            lanes = lax.iota(jnp.int32, 16)
            nxt_idx = jnp.minimum(lanes + 1, 15)
            gdn = lax.GatherDimensionNumbers(
                offset_dims=(), collapsed_slice_dims=(0,), start_index_map=(0,)
            )

            def chunk(ci, carry):
                base = (ci * 16 + lanes) * 4
                e0 = plsc.load_gather(xloc, [base])
                e1 = plsc.load_gather(xloc, [base + 1])
                ev = plsc.load_gather(xloc, [base + 2])
                et = plsc.load_gather(xloc, [base + 3])
                feat = (e0 * e1).astype(jnp.int32)
                ts = et.astype(jnp.int32)
                key = ts * _IN_F + feat
                sortk = key * 16 + lanes
                sk, sv = plsc.sort_key_val(sortk, ev)
                key_s = lax.shift_right_logical(sk, 4)
                nxt = lax.gather(
                    key_s,
                    nxt_idx[:, None],
                    gdn,
                    (1,),
                    mode=lax.GatherScatterMode.PROMISE_IN_BOUNDS,
                )
                keep = ((key_s != nxt) | (lanes == 15)) & (key_s < _ZWORDS)
                ts_s = lax.shift_right_logical(key_s, 10)
                ft_s = lax.bitwise_and(key_s, _IN_F - 1)
                plsc.store_scatter(zloc, [ts_s, ft_s], sv, mask=keep)
                return carry

            lax.fori_loop(0, _S // 16, chunk, 0)
            pltpu.sync_copy(zloc, out_hbm.at[:, b])

    return k(x.reshape(_B, _S * 4))


_NCHUNK = 32
_CROWS = _HID_F // _NCHUNK
_WICHUNK = 16
_WIROWS = _HID_F // _WICHUNK


def _rnn_kernel(
    zt_ref,
    w_in_hbm,
    w_rec_hbm,
    w_out_ref,
    out_ref,
    i_in_s,
    wrec16,
    zh_s,
    z_s,
    v_s,
    i_s,
    y_s,
    vo_s,
    io_s,
    wtmp0,
    wtmp1,
    witmp0,
    witmp1,
    wch16,
    zt16_s,
    sem0,
    sem1,
    wisem0,
    wisem1,
):
    bufs = (wtmp0, wtmp1)
    sems = (sem0, sem1)
    wibufs = (witmp0, witmp1)
    wisems = (wisem0, wisem1)
    pltpu.make_async_copy(w_in_hbm.at[pl.ds(0, _WIROWS), :], wibufs[0], wisems[0]).start()
    pltpu.make_async_copy(w_in_hbm.at[pl.ds(_WIROWS, _WIROWS), :], wibufs[1], wisems[1]).start()
    pltpu.make_async_copy(w_rec_hbm.at[pl.ds(0, _CROWS), :], bufs[0], sems[0]).start()
    pltpu.make_async_copy(w_rec_hbm.at[pl.ds(_CROWS, _CROWS), :], bufs[1], sems[1]).start()

    def wrec_chunk(k):
        pltpu.make_async_copy(
            w_rec_hbm.at[pl.ds(k * _CROWS, _CROWS), :], bufs[k % 2], sems[k % 2]
        ).wait()
        wrec16[pl.ds(k * _CROWS, _CROWS), :] = bufs[k % 2][...].astype(jnp.bfloat16)
        if k + 2 < _NCHUNK:
            pltpu.make_async_copy(
                w_rec_hbm.at[pl.ds((k + 2) * _CROWS, _CROWS), :],
                bufs[k % 2],
                sems[k % 2],
            ).start()

    zt16_s[...] = zt_ref[...].astype(jnp.bfloat16)
    for k in range(_WICHUNK):
        pltpu.make_async_copy(
            w_in_hbm.at[pl.ds(k * _WIROWS, _WIROWS), :], wibufs[k % 2], wisems[k % 2]
        ).wait()
        wch16[...] = wibufs[k % 2][...].astype(jnp.bfloat16)
        if k + 2 < _WICHUNK:
            pltpu.make_async_copy(
                w_in_hbm.at[pl.ds((k + 2) * _WIROWS, _WIROWS), :],
                wibufs[k % 2],
                wisems[k % 2],
            ).start()
        i_in_s[:, pl.ds(k * _WIROWS, _WIROWS)] = lax.dot_general(
            zt16_s[...],
            wch16[...],
            (((1,), (1,)), ((), ())),
            preferred_element_type=jnp.float32,
        )
        wrec_chunk(k)

    z_s[...] = jnp.zeros_like(z_s)
    v_s[...] = jnp.zeros_like(v_s)
    i_s[...] = jnp.zeros_like(i_s)

    def warm_step(t):
        v = v_s[...]
        i = i_s[...]
        v_dec = v + _DT * _TAU_MEM_INV * ((0.0 - v) + i)
        i_dec = i - _DT * _TAU_SYN_INV * i
        z_new = ((v_dec - _V_TH) > 0.0).astype(jnp.float32)
        v_s[...] = (1.0 - z_new) * v_dec
        i_s[...] = i_dec + i_in_s[pl.ds(t * _B, _B), :]
        z16 = z_new.astype(jnp.bfloat16)
        z_s[...] = z16
        zh_s[pl.ds(t * _B, _B), :] = z16

    warm_step(0)
    warm_step(1)

    for k in range(_WICHUNK, _NCHUNK):
        wrec_chunk(k)

    def step(t, carry):
        v = v_s[...]
        i = i_s[...]
        v_dec = v + _DT * _TAU_MEM_INV * ((0.0 - v) + i)
        i_dec = i - _DT * _TAU_SYN_INV * i
        z_new = ((v_dec - _V_TH) > 0.0).astype(jnp.float32)
        v_s[...] = (1.0 - z_new) * v_dec
        rec = lax.dot_general(
            z_s[...],
            wrec16[...],
            (((1,), (0,)), ((), ())),
            preferred_element_type=jnp.float32,
        )
        i_s[...] = (i_dec + i_in_s[pl.ds(t * _B, _B), :]) + rec
        z16 = z_new.astype(jnp.bfloat16)
        z_s[...] = z16
        zh_s[pl.ds(t * _B, _B), :] = z16
        return carry

    lax.fori_loop(2, _T, step, 0)

    y_s[...] = lax.dot_general(
        zh_s[...],
        w_out_ref[...],
        (((1,), (1,)), ((), ())),
        preferred_element_type=jnp.float32,
    )

    vo_s[...] = jnp.zeros_like(vo_s)
    io_s[...] = jnp.zeros_like(io_s)

    def li_step(t, carry):
        vo = vo_s[...]
        io = io_s[...]
        y = y_s[pl.ds(t * _B, _B), :]
        vo_new = vo + _DT * _TAU_MEM_INV * ((0.0 - vo) + io)
        io_s[...] = io - _DT * _TAU_SYN_INV * io + y
        vo_s[...] = vo_new
        out_ref[pl.ds(t * _B, _B), :] = vo_new
        return carry

    lax.fori_loop(0, _T, li_step, 0)


def kernel(x, W_in, W_rec, W_out):
    z = _sc_scatter(x)
    zt = z.reshape(_T * _B, _IN_F)
    v_out = pl.pallas_call(
        _rnn_kernel,
        out_shape=jax.ShapeDtypeStruct((_T * _B, _OUT_F), jnp.float32),
        in_specs=[
            pl.BlockSpec(memory_space=pltpu.MemorySpace.VMEM),
            pl.BlockSpec(memory_space=pltpu.MemorySpace.HBM),
            pl.BlockSpec(memory_space=pltpu.MemorySpace.HBM),
            pl.BlockSpec(memory_space=pltpu.MemorySpace.VMEM),
        ],
        scratch_shapes=[
            pltpu.VMEM((_T * _B, _HID_F), jnp.float32),
            pltpu.VMEM((_HID_F, _HID_F), jnp.bfloat16),
            pltpu.VMEM((_T * _B, _HID_F), jnp.bfloat16),
            pltpu.VMEM((_B, _HID_F), jnp.bfloat16),
            pltpu.VMEM((_B, _HID_F), jnp.float32),
            pltpu.VMEM((_B, _HID_F), jnp.float32),
            pltpu.VMEM((_T * _B, _OUT_F), jnp.float32),
            pltpu.VMEM((_B, _OUT_F), jnp.float32),
            pltpu.VMEM((_B, _OUT_F), jnp.float32),
            pltpu.VMEM((_CROWS, _HID_F), jnp.float32),
            pltpu.VMEM((_CROWS, _HID_F), jnp.float32),
            pltpu.VMEM((_WIROWS, _IN_F), jnp.float32),
            pltpu.VMEM((_WIROWS, _IN_F), jnp.float32),
            pltpu.VMEM((_WIROWS, _IN_F), jnp.bfloat16),
            pltpu.VMEM((_T * _B, _IN_F), jnp.bfloat16),
            pltpu.SemaphoreType.DMA,
            pltpu.SemaphoreType.DMA,
            pltpu.SemaphoreType.DMA,
            pltpu.SemaphoreType.DMA,
        ],
        compiler_params=pltpu.CompilerParams(vmem_limit_bytes=58 * 1024 * 1024),
    )(zt, W_in, jnp.swapaxes(W_rec, 0, 1), W_out.astype(jnp.bfloat16))
    return v_out.reshape(_T, _B, _OUT_F)


# speedup vs baseline: 1.6647x; 1.2275x over previous
import functools

import jax
import jax.numpy as jnp
from jax import lax
from jax.experimental import pallas as pl
from jax.experimental.pallas import tpu as pltpu
from jax.experimental.pallas import tpu_sc as plsc

_DT = 0.001
_TAU_SYN_INV = 200.0
_TAU_MEM_INV = 100.0
_V_TH = 0.5

_B, _S = 16, 2048
_IN_F, _HID_F, _OUT_F = 1024, 4096, 256
_T = 31
_ZWORDS = _T * _IN_F


def _sc_scatter(x):
    mesh = plsc.VectorSubcoreMesh(core_axis_name="c", subcore_axis_name="s")

    @functools.partial(
        pl.kernel,
        mesh=mesh,
        compiler_params=pltpu.CompilerParams(needs_layout_passes=False),
        out_type=jax.ShapeDtypeStruct((_T, _B, _IN_F), jnp.float32),
        scratch_types=[
            pltpu.VMEM((_S * 4,), jnp.float32),
            pltpu.VMEM((_T, _IN_F), jnp.float32),
        ],
    )
    def k(x_hbm, out_hbm, xloc, zloc):
        cid = lax.axis_index("c")
        sid = lax.axis_index("s")
        wid = sid * 2 + cid

        @pl.when(wid < _B)
        def _():
            b = wid
            pltpu.sync_copy(x_hbm.at[b], xloc)

            zero16 = jnp.zeros((16,), jnp.float32)

            def zero_row(r, carry):
                for c in range(_IN_F // 16):
                    zloc[r, pl.ds(c * 16, 16)] = zero16
                return carry

            lax.fori_loop(0, _T, zero_row, 0)

            lanes = lax.iota(jnp.int32, 16)
            nxt_idx = jnp.minimum(lanes + 1, 15)
            gdn = lax.GatherDimensionNumbers(
                offset_dims=(), collapsed_slice_dims=(0,), start_index_map=(0,)
            )

            def chunk(ci, carry):
                base = (ci * 16 + lanes) * 4
                e0 = plsc.load_gather(xloc, [base])
                e1 = plsc.load_gather(xloc, [base + 1])
                ev = plsc.load_gather(xloc, [base + 2])
                et = plsc.load_gather(xloc, [base + 3])
                feat = (e0 * e1).astype(jnp.int32)
                ts = et.astype(jnp.int32)
                key = ts * _IN_F + feat
                sortk = key * 16 + lanes
                sk, sv = plsc.sort_key_val(sortk, ev)
                key_s = lax.shift_right_logical(sk, 4)
                nxt = lax.gather(
                    key_s,
                    nxt_idx[:, None],
                    gdn,
                    (1,),
                    mode=lax.GatherScatterMode.PROMISE_IN_BOUNDS,
                )
                keep = ((key_s != nxt) | (lanes == 15)) & (key_s < _ZWORDS)
                ts_s = lax.shift_right_logical(key_s, 10)
                ft_s = lax.bitwise_and(key_s, _IN_F - 1)
                plsc.store_scatter(zloc, [ts_s, ft_s], sv, mask=keep)
                return carry

            lax.fori_loop(0, _S // 16, chunk, 0)
            pltpu.sync_copy(zloc, out_hbm.at[:, b])

    return k(x.reshape(_B, _S * 4))


_NCHUNK = 32
_CROWS = _HID_F // _NCHUNK
_WICHUNK = 16
_WIROWS = _HID_F // _WICHUNK


def _rnn_kernel(
    zt_ref,
    w_in_hbm,
    w_rec_hbm,
    w_out_ref,
    out_ref,
    i_in_s,
    wrec16,
    zh_s,
    z_s,
    v_s,
    i_s,
    y_s,
    vo_s,
    io_s,
    wtmp0,
    wtmp1,
    witmp0,
    witmp1,
    wch16,
    zt16_s,
    sem0,
    sem1,
    wisem0,
    wisem1,
):
    bufs = (wtmp0, wtmp1)
    sems = (sem0, sem1)
    wibufs = (witmp0, witmp1)
    wisems = (wisem0, wisem1)
    pltpu.make_async_copy(w_in_hbm.at[pl.ds(0, _WIROWS), :], wibufs[0], wisems[0]).start()
    pltpu.make_async_copy(w_in_hbm.at[pl.ds(_WIROWS, _WIROWS), :], wibufs[1], wisems[1]).start()
    pltpu.make_async_copy(w_rec_hbm.at[pl.ds(0, _CROWS), :], bufs[0], sems[0]).start()
    pltpu.make_async_copy(w_rec_hbm.at[pl.ds(_CROWS, _CROWS), :], bufs[1], sems[1]).start()

    def wrec_chunk(k):
        pltpu.make_async_copy(
            w_rec_hbm.at[pl.ds(k * _CROWS, _CROWS), :], bufs[k % 2], sems[k % 2]
        ).wait()
        for p in range(4):
            blk = bufs[k % 2][:, pl.ds(p * (_HID_F // 4), _HID_F // 4)]
            wrec16[pl.ds(p * (_HID_F // 4), _HID_F // 4), pl.ds(k * _CROWS, _CROWS)] = (
                jnp.swapaxes(blk.astype(jnp.bfloat16), 0, 1)
            )
        if k + 2 < _NCHUNK:
            pltpu.make_async_copy(
                w_rec_hbm.at[pl.ds((k + 2) * _CROWS, _CROWS), :],
                bufs[k % 2],
                sems[k % 2],
            ).start()

    zt16_s[...] = zt_ref[...].astype(jnp.bfloat16)
    for k in range(_WICHUNK):
        pltpu.make_async_copy(
            w_in_hbm.at[pl.ds(k * _WIROWS, _WIROWS), :], wibufs[k % 2], wisems[k % 2]
        ).wait()
        wch16[...] = wibufs[k % 2][...].astype(jnp.bfloat16)
        if k + 2 < _WICHUNK:
            pltpu.make_async_copy(
                w_in_hbm.at[pl.ds((k + 2) * _WIROWS, _WIROWS), :],
                wibufs[k % 2],
                wisems[k % 2],
            ).start()
        i_in_s[:, pl.ds(k * _WIROWS, _WIROWS)] = lax.dot_general(
            zt16_s[...],
            wch16[...],
            (((1,), (1,)), ((), ())),
            preferred_element_type=jnp.float32,
        )
        wrec_chunk(k)

    z_s[...] = jnp.zeros_like(z_s)
    v_s[...] = jnp.zeros_like(v_s)
    i_s[...] = jnp.zeros_like(i_s)

    def warm_step(t):
        v = v_s[...]
        i = i_s[...]
        v_dec = v + _DT * _TAU_MEM_INV * ((0.0 - v) + i)
        i_dec = i - _DT * _TAU_SYN_INV * i
        z_new = ((v_dec - _V_TH) > 0.0).astype(jnp.float32)
        v_s[...] = (1.0 - z_new) * v_dec
        i_s[...] = i_dec + i_in_s[pl.ds(t * _B, _B), :]
        z16 = z_new.astype(jnp.bfloat16)
        z_s[...] = z16
        zh_s[pl.ds(t * _B, _B), :] = z16

    warm_step(0)
    warm_step(1)

    for k in range(_WICHUNK, _NCHUNK):
        wrec_chunk(k)

    def step(t, carry):
        v = v_s[...]
        i = i_s[...]
        v_dec = v + _DT * _TAU_MEM_INV * ((0.0 - v) + i)
        i_dec = i - _DT * _TAU_SYN_INV * i
        z_new = ((v_dec - _V_TH) > 0.0).astype(jnp.float32)
        v_s[...] = (1.0 - z_new) * v_dec
        rec = lax.dot_general(
            z_s[...],
            wrec16[...],
            (((1,), (0,)), ((), ())),
            preferred_element_type=jnp.float32,
        )
        i_s[...] = (i_dec + i_in_s[pl.ds(t * _B, _B), :]) + rec
        z16 = z_new.astype(jnp.bfloat16)
        z_s[...] = z16
        zh_s[pl.ds(t * _B, _B), :] = z16
        return carry

    lax.fori_loop(2, _T, step, 0)

    y_s[...] = lax.dot_general(
        zh_s[...],
        w_out_ref[...],
        (((1,), (1,)), ((), ())),
        preferred_element_type=jnp.float32,
    )

    vo_s[...] = jnp.zeros_like(vo_s)
    io_s[...] = jnp.zeros_like(io_s)

    def li_step(t, carry):
        vo = vo_s[...]
        io = io_s[...]
        y = y_s[pl.ds(t * _B, _B), :]
        vo_new = vo + _DT * _TAU_MEM_INV * ((0.0 - vo) + io)
        io_s[...] = io - _DT * _TAU_SYN_INV * io + y
        vo_s[...] = vo_new
        out_ref[pl.ds(t * _B, _B), :] = vo_new
        return carry

    lax.fori_loop(0, _T, li_step, 0)


def kernel(x, W_in, W_rec, W_out):
    z = _sc_scatter(x)
    zt = z.reshape(_T * _B, _IN_F)
    v_out = pl.pallas_call(
        _rnn_kernel,
        out_shape=jax.ShapeDtypeStruct((_T * _B, _OUT_F), jnp.float32),
        in_specs=[
            pl.BlockSpec(memory_space=pltpu.MemorySpace.VMEM),
            pl.BlockSpec(memory_space=pltpu.MemorySpace.HBM),
            pl.BlockSpec(memory_space=pltpu.MemorySpace.HBM),
            pl.BlockSpec(memory_space=pltpu.MemorySpace.VMEM),
        ],
        scratch_shapes=[
            pltpu.VMEM((_T * _B, _HID_F), jnp.float32),
            pltpu.VMEM((_HID_F, _HID_F), jnp.bfloat16),
            pltpu.VMEM((_T * _B, _HID_F), jnp.bfloat16),
            pltpu.VMEM((_B, _HID_F), jnp.bfloat16),
            pltpu.VMEM((_B, _HID_F), jnp.float32),
            pltpu.VMEM((_B, _HID_F), jnp.float32),
            pltpu.VMEM((_T * _B, _OUT_F), jnp.float32),
            pltpu.VMEM((_B, _OUT_F), jnp.float32),
            pltpu.VMEM((_B, _OUT_F), jnp.float32),
            pltpu.VMEM((_CROWS, _HID_F), jnp.float32),
            pltpu.VMEM((_CROWS, _HID_F), jnp.float32),
            pltpu.VMEM((_WIROWS, _IN_F), jnp.float32),
            pltpu.VMEM((_WIROWS, _IN_F), jnp.float32),
            pltpu.VMEM((_WIROWS, _IN_F), jnp.bfloat16),
            pltpu.VMEM((_T * _B, _IN_F), jnp.bfloat16),
            pltpu.SemaphoreType.DMA,
            pltpu.SemaphoreType.DMA,
            pltpu.SemaphoreType.DMA,
            pltpu.SemaphoreType.DMA,
        ],
        compiler_params=pltpu.CompilerParams(vmem_limit_bytes=58 * 1024 * 1024),
    )(zt, W_in, W_rec, W_out.astype(jnp.bfloat16))
    return v_out.reshape(_T, _B, _OUT_F)
